# baseline (speedup 1.0000x reference)
import functools

import jax
import jax.numpy as jnp
from jax import lax
from jax.experimental import pallas as pl
from jax.experimental.pallas import tpu as pltpu
from jax.experimental.pallas import tpu_sc as plsc

B, S, D = 8, 4096, 1024
NC, NS = 2, 16
NW = NC * NS
ROWS_PER_W = S // NW
CH = 32
NCHUNK = ROWS_PER_W // CH
PH = 16
L = 16


def _body(seg_hbm, table_hbm, out_hbm,
          t0_v, t1_v, z_v, p_v, s0_v, s1_v,
          sem_l, sem_s, sem_w):
    wid = lax.axis_index("s") * NC + lax.axis_index("c")
    base = wid * ROWS_PER_W
    tbufs = [t0_v, t1_v]
    sbufs = [s0_v, s1_v]

    pltpu.async_copy(table_hbm.at[pl.ds(base + 2, CH)], t0_v, sem_l)
    pltpu.async_copy(seg_hbm.at[0], s0_v, sem_s)

    zvec = jnp.zeros((L,), jnp.float32)

    def zrow(r, _):
        def zcol(k, _):
            z_v[r, pl.ds(k * 4 * L, L)] = zvec
            z_v[r, pl.ds((k * 4 + 1) * L, L)] = zvec
            z_v[r, pl.ds((k * 4 + 2) * L, L)] = zvec
            z_v[r, pl.ds((k * 4 + 3) * L, L)] = zvec
            return 0
        return lax.fori_loop(0, D // L // 4, zcol, 0)

    lax.fori_loop(0, CH, zrow, 0)

    counts = []
    for i in range(B):
        sv = sbufs[i % 2]
        pltpu.make_async_copy(seg_hbm.at[i], sv, sem_s).wait()
        if i + 1 < B:
            pltpu.async_copy(seg_hbm.at[i + 1],
                             sbufs[(i + 1) % 2], sem_s)

        def red(k, acc):
            b0 = k * 4 * L
            acc = acc + sv[pl.ds(b0, L)]
            acc = acc + sv[pl.ds(b0 + L, L)]
            acc = acc + sv[pl.ds(b0 + 2 * L, L)]
            acc = acc + sv[pl.ds(b0 + 3 * L, L)]
            return acc

        acc = lax.fori_loop(0, S // L // 4, red, jnp.zeros((L,), jnp.int32))
        counts.append(jnp.sum(acc))

    def drain_writes(n):
        def w(_, carry):
            pltpu.make_async_copy(z_v, out_hbm.at[0, pl.ds(0, CH)], sem_w).wait()
            return carry
        lax.fori_loop(0, n, w, 0)

    fired = jnp.int32(0)
    drained = jnp.int32(0)

    for q in range(NCHUNK):
        t_v = tbufs[q % 2]
        j0 = base + q * CH
        pltpu.make_async_copy(table_hbm.at[pl.ds(j0 + 2, CH)], t_v,
                              sem_l).wait()
        if q + 1 < NCHUNK:
            drain_writes(fired - drained)
            drained = fired
            pltpu.async_copy(table_hbm.at[pl.ds(j0 + CH + 2, CH)],
                             tbufs[(q + 1) % 2], sem_l)

        for i in range(B):
            c = counts[i]

            full = (j0 + CH) <= c
            empty = j0 >= c
            boundary = jnp.logical_and(jnp.logical_not(full),
                                       jnp.logical_not(empty))

            @pl.when(full)
            def _():
                pltpu.async_copy(t_v, out_hbm.at[i, pl.ds(j0, CH)], sem_w)

            @pl.when(empty)
            def _():
                pltpu.async_copy(z_v, out_hbm.at[i, pl.ds(j0, CH)], sem_w)

            fired = fired + jnp.where(boundary, 0, 1).astype(jnp.int32)

            @pl.when(jnp.logical_and(boundary, c >= CH))
            def _():
                pltpu.sync_copy(z_v, out_hbm.at[i, pl.ds(j0, CH)])
                s0 = c - CH
                for h in range(CH // PH):
                    pltpu.sync_copy(
                        table_hbm.at[pl.ds(s0 + h * PH + 2, PH)], p_v)
                    pltpu.sync_copy(
                        p_v, out_hbm.at[i, pl.ds(s0 + h * PH, PH)])

            @pl.when(jnp.logical_and(boundary, c < CH))
            def _():
                pltpu.sync_copy(z_v, out_hbm.at[i, pl.ds(j0, CH)])

                def rowloop(r, _):
                    @pl.when(r < c)
                    def _():
                        pltpu.sync_copy(t_v.at[pl.ds(r, 1)],
                                        out_hbm.at[i, pl.ds(j0 + r, 1)])
                    return 0

                lax.fori_loop(0, CH, rowloop, 0)

    drain_writes(fired - drained)


@functools.partial(jax.jit, static_argnames=())
def _run(seg, emb_table):
    mesh = plsc.VectorSubcoreMesh(core_axis_name="c", subcore_axis_name="s")
    f = pl.kernel(
        _body,
        out_type=jax.ShapeDtypeStruct((B, S, D), jnp.float32),
        mesh=mesh,
        scratch_types=[
            pltpu.VMEM((CH, D), jnp.float32),
            pltpu.VMEM((CH, D), jnp.float32),
            pltpu.VMEM((CH, D), jnp.float32),
            pltpu.VMEM((PH, D), jnp.float32),
            pltpu.VMEM((S,), jnp.int32),
            pltpu.VMEM((S,), jnp.int32),
            pltpu.SemaphoreType.DMA,
            pltpu.SemaphoreType.DMA,
            pltpu.SemaphoreType.DMA,
        ],
        compiler_params=pltpu.CompilerParams(use_tc_tiling_on_sc=False,
                                             needs_layout_passes=False),
    )
    return f(seg, emb_table)


def kernel(src, seg, emb_table):
    del src
    return _run(seg, emb_table)


# speedup vs baseline: 2.3642x; 2.3642x over previous
import functools

import jax
import jax.numpy as jnp
from jax import lax
from jax.experimental import pallas as pl
from jax.experimental.pallas import tpu as pltpu
from jax.experimental.pallas import tpu_sc as plsc

B, S, D = 8, 4096, 1024
NC, NS = 2, 16
NW = NC * NS
ROWS_PER_W = S // NW
CH = 32
NCHUNK = ROWS_PER_W // CH
ZH = 16
PH = 16
SSTRIP = 512
NSTRIP = S // SSTRIP
L = 16


def _body(seg_hbm, table_hbm, out_hbm,
          t0_v, t1_v, z_v, p_v, sg0_v, sg1_v, idx0_v, idx1_v, pidx_v,
          sem_l, sem_s, sem_w, sem_p):
    wid = lax.axis_index("s") * NC + lax.axis_index("c")
    tbufs = [t0_v, t1_v]
    idxbufs = [idx0_v, idx1_v]
    sgbufs = [sg0_v, sg1_v]
    lane = lax.iota(jnp.int32, L)

    def chunk_j0(q):
        return (q * NW + wid) * CH

    def build_idx(ref, j0):
        ref[pl.ds(0, L)] = lane + (j0 + 2)
        ref[pl.ds(L, L)] = lane + (j0 + 2 + L)

    build_idx(idx0_v, chunk_j0(0))
    pltpu.async_copy(table_hbm.at[idx0_v], t0_v, sem_l)
    pltpu.async_copy(seg_hbm.at[:, pl.ds(0, SSTRIP)], sg0_v, sem_s)

    zvec = jnp.zeros((L,), jnp.float32)

    def zrow(r, _):
        def zcol(k, _):
            z_v[r, pl.ds(k * 4 * L, L)] = zvec
            z_v[r, pl.ds((k * 4 + 1) * L, L)] = zvec
            z_v[r, pl.ds((k * 4 + 2) * L, L)] = zvec
            z_v[r, pl.ds((k * 4 + 3) * L, L)] = zvec
            return 0
        return lax.fori_loop(0, D // L // 4, zcol, 0)

    lax.fori_loop(0, ZH, zrow, 0)

    accs = tuple(jnp.zeros((L,), jnp.int32) for _ in range(B))
    for k in range(NSTRIP):
        sg = sgbufs[k % 2]
        pltpu.make_async_copy(
            seg_hbm.at[:, pl.ds(k * SSTRIP, SSTRIP)], sg, sem_s).wait()
        if k + 1 < NSTRIP:
            pltpu.async_copy(
                seg_hbm.at[:, pl.ds((k + 1) * SSTRIP, SSTRIP)],
                sgbufs[(k + 1) % 2], sem_s)

        def red(t, acc):
            return tuple(acc[i] + sg[i, pl.ds(t * L, L)] for i in range(B))

        accs = lax.fori_loop(0, SSTRIP // L, red, accs)
    counts = [jnp.sum(a) for a in accs]

    def drain_writes(n):
        def w(_, carry):
            pltpu.make_async_copy(z_v, out_hbm.at[0, pl.ds(0, ZH)],
                                  sem_w).wait()
            return carry
        lax.fori_loop(0, n, w, 0)

    fired = jnp.int32(0)
    drained = jnp.int32(0)

    for q in range(NCHUNK):
        t_v = tbufs[q % 2]
        j0 = chunk_j0(q)
        pltpu.make_async_copy(table_hbm.at[idxbufs[q % 2]], t_v,
                              sem_l).wait()
        if q + 1 < NCHUNK:
            drain_writes(fired - drained)
            drained = fired
            build_idx(idxbufs[(q + 1) % 2], chunk_j0(q + 1))
            pltpu.async_copy(table_hbm.at[idxbufs[(q + 1) % 2]],
                             tbufs[(q + 1) % 2], sem_l)

        for i in range(B):
            c = counts[i]
            full = (j0 + CH) <= c
            empty = j0 >= c
            boundary = jnp.logical_and(jnp.logical_not(full),
                                       jnp.logical_not(empty))

            @pl.when(full)
            def _():
                pltpu.async_copy(t_v, out_hbm.at[i, pl.ds(j0, CH)], sem_w)

            @pl.when(empty)
            def _():
                pltpu.async_copy(z_v, out_hbm.at[i, pl.ds(j0, ZH)], sem_w)
                pltpu.async_copy(z_v, out_hbm.at[i, pl.ds(j0 + ZH, ZH)],
                                 sem_w)

            fired = fired + jnp.where(boundary, 0, 2).astype(jnp.int32)

            @pl.when(boundary)
            def _():
                for h in range(CH // PH):
                    jv = lane + (j0 + h * PH)
                    pidx_v[pl.ds(0, L)] = jnp.where(jv < c, jv + 2, 0)
                    pltpu.async_copy(table_hbm.at[pidx_v], p_v,
                                     sem_p).wait()
                    pltpu.sync_copy(p_v,
                                    out_hbm.at[i, pl.ds(j0 + h * PH, PH)])

    drain_writes(fired - drained)


@functools.partial(jax.jit, static_argnames=())
def _run(seg, emb_table):
    mesh = plsc.VectorSubcoreMesh(core_axis_name="c", subcore_axis_name="s")
    f = pl.kernel(
        _body,
        out_type=jax.ShapeDtypeStruct((B, S, D), jnp.float32),
        mesh=mesh,
        scratch_types=[
            pltpu.VMEM((CH, D), jnp.float32),
            pltpu.VMEM((CH, D), jnp.float32),
            pltpu.VMEM((ZH, D), jnp.float32),
            pltpu.VMEM((PH, D), jnp.float32),
            pltpu.VMEM((B, SSTRIP), jnp.int32),
            pltpu.VMEM((B, SSTRIP), jnp.int32),
            pltpu.VMEM((CH,), jnp.int32),
            pltpu.VMEM((CH,), jnp.int32),
            pltpu.VMEM((PH,), jnp.int32),
            pltpu.SemaphoreType.DMA,
            pltpu.SemaphoreType.DMA,
            pltpu.SemaphoreType.DMA,
            pltpu.SemaphoreType.DMA,
        ],
        compiler_params=pltpu.CompilerParams(needs_layout_passes=False),
    )
    return f(seg, emb_table)


def kernel(src, seg, emb_table):
    del src
    return _run(seg, emb_table)
